# baseline (speedup 1.0000x reference)
import functools

import jax
import jax.numpy as jnp
from jax import lax
from jax.experimental import pallas as pl
from jax.experimental.pallas import tpu as pltpu
from jax.experimental.pallas import tpu_sc as plsc

_N_SLOW = 8
_N_FAST = 32
_BC = 12
_NC = 2
_NS = 16
_NW = _NC * _NS
_FAST_PER_W = (_BC * _N_FAST) // _NW
_HALF = 112
_DEPTH = 4
_CHUNKS = _FAST_PER_W * 2


def _sc_fast_body(frames_hbm, fast_hbm, bufs, sems_in, sems_out):
    cid = lax.axis_index("c")
    sid = lax.axis_index("s")
    wid = sid * _NC + cid

    def chunk(j):
        p, half = j // 2, j % 2
        s = wid * _FAST_PER_W + p
        bc = s // _N_FAST
        t = s % _N_FAST
        src_t = 2 * t + jnp.where(t == _N_FAST - 1, 1, 0)
        rows = pl.ds(half * _HALF, _HALF)
        return frames_hbm.at[bc, src_t, rows], fast_hbm.at[bc, t, rows]

    pending_rd = [None] * _DEPTH
    pending_wr = [None] * _DEPTH
    for j in range(_DEPTH - 1):
        k = j % _DEPTH
        src, _ = chunk(j)
        pending_rd[k] = pltpu.make_async_copy(src, bufs.at[k], sems_in.at[k])
        pending_rd[k].start()
    for j in range(_CHUNKS):
        k = j % _DEPTH
        jn = j + _DEPTH - 1
        if jn < _CHUNKS:
            kn = jn % _DEPTH
            if pending_wr[kn] is not None:
                pending_wr[kn].wait()
            src_n, _ = chunk(jn)
            pending_rd[kn] = pltpu.make_async_copy(src_n, bufs.at[kn],
                                                   sems_in.at[kn])
            pending_rd[kn].start()
        pending_rd[k].wait()
        _, dst = chunk(j)
        wr = pltpu.make_async_copy(bufs.at[k], dst, sems_out.at[k])
        wr.start()
        pending_wr[k] = wr
    for j in range(_CHUNKS - _DEPTH + 1, _CHUNKS):
        pending_wr[j % _DEPTH].wait()
    pending_wr[(_CHUNKS - _DEPTH) % _DEPTH].wait()


def _tc_slow_body(frames_ref, slow_ref):
    slow_ref[...] = frames_ref[...]


def kernel(frames):
    B, C, T, H, W = frames.shape
    frames3d = frames.reshape(B * C, T, H, W)

    mesh = plsc.VectorSubcoreMesh(core_axis_name="c", subcore_axis_name="s",
                                  num_cores=_NC, num_subcores=_NS)
    run_fast = functools.partial(
        pl.kernel,
        out_type=jax.ShapeDtypeStruct((_BC, _N_FAST, H, W), jnp.float32),
        mesh=mesh,
        scratch_types=[
            pltpu.VMEM((_DEPTH, _HALF, W), jnp.float32),
            pltpu.SemaphoreType.DMA((_DEPTH,)),
            pltpu.SemaphoreType.DMA((_DEPTH,)),
        ],
        compiler_params=pltpu.CompilerParams(use_tc_tiling_on_sc=True),
    )(_sc_fast_body)
    fast4d = run_fast(frames3d)

    slow4d = pl.pallas_call(
        _tc_slow_body,
        grid=(_N_SLOW,),
        in_specs=[pl.BlockSpec((_BC, 1, H, W), lambda t: (0, 9 * t, 0, 0))],
        out_specs=pl.BlockSpec((_BC, 1, H, W), lambda t: (0, t, 0, 0)),
        out_shape=jax.ShapeDtypeStruct((_BC, _N_SLOW, H, W), jnp.float32),
    )(frames3d)

    slow = slow4d.reshape(B, C, _N_SLOW, H, W)
    fast = fast4d.reshape(B, C, _N_FAST, H, W)
    return (slow, fast)


# speedup vs baseline: 1.0071x; 1.0071x over previous
import functools

import jax
import jax.numpy as jnp
from jax import lax
from jax.experimental import pallas as pl
from jax.experimental.pallas import tpu as pltpu
from jax.experimental.pallas import tpu_sc as plsc

_N_SLOW = 8
_N_FAST = 32
_BC = 12
_NC = 2
_NS = 16
_NW = _NC * _NS
_SLOW_PER_W = (_BC * _N_SLOW) // _NW


def _sc_slow_body(frames_hbm, slow_hbm,
                  buf0, buf1, sem_in0, sem_in1, sem_out0, sem_out1):
    cid = lax.axis_index("c")
    sid = lax.axis_index("s")
    wid = sid * _NC + cid

    def plane(j):
        s = wid * _SLOW_PER_W + j
        bc = s // _N_SLOW
        t = s % _N_SLOW
        return frames_hbm.at[bc, t * 9], slow_hbm.at[bc, t]

    bufs = (buf0, buf1)
    sems_in = (sem_in0, sem_in1)
    sems_out = (sem_out0, sem_out1)
    pending_wr = [None, None]
    pending_rd = [None, None]

    src0, _ = plane(0)
    pending_rd[0] = pltpu.make_async_copy(src0, buf0, sem_in0)
    pending_rd[0].start()
    for j in range(_SLOW_PER_W):
        k = j % 2
        kn = (j + 1) % 2
        if j + 1 < _SLOW_PER_W:
            if pending_wr[kn] is not None:
                pending_wr[kn].wait()
            src_n, _ = plane(j + 1)
            pending_rd[kn] = pltpu.make_async_copy(src_n, bufs[kn],
                                                   sems_in[kn])
            pending_rd[kn].start()
        pending_rd[k].wait()
        _, dst = plane(j)
        wr = pltpu.make_async_copy(bufs[k], dst, sems_out[k])
        wr.start()
        pending_wr[k] = wr
    pending_wr[0].wait()
    pending_wr[1].wait()


def _tc_fast_body(frames_ref, fast_ref):
    fast_ref[...] = frames_ref[...]


def kernel(frames):
    B, C, T, H, W = frames.shape
    frames3d = frames.reshape(B * C, T, H, W)

    mesh = plsc.VectorSubcoreMesh(core_axis_name="c", subcore_axis_name="s",
                                  num_cores=_NC, num_subcores=_NS)
    run_slow = functools.partial(
        pl.kernel,
        out_type=jax.ShapeDtypeStruct((_BC, _N_SLOW, H, W), jnp.float32),
        mesh=mesh,
        scratch_types=[
            pltpu.VMEM((H, W), jnp.float32),
            pltpu.VMEM((H, W), jnp.float32),
            pltpu.SemaphoreType.DMA,
            pltpu.SemaphoreType.DMA,
            pltpu.SemaphoreType.DMA,
            pltpu.SemaphoreType.DMA,
        ],
        compiler_params=pltpu.CompilerParams(use_tc_tiling_on_sc=True, skip_device_barrier=True),
    )(_sc_slow_body)

    fast4d = pl.pallas_call(
        _tc_fast_body,
        grid=(_N_FAST,),
        in_specs=[pl.BlockSpec(
            (_BC, 1, H, W),
            lambda t: (0, 2 * t + jnp.where(t == _N_FAST - 1, 1, 0), 0, 0))],
        out_specs=pl.BlockSpec((_BC, 1, H, W), lambda t: (0, t, 0, 0)),
        out_shape=jax.ShapeDtypeStruct((_BC, _N_FAST, H, W), jnp.float32),
    )(frames3d)
    slow4d = run_slow(frames3d)

    slow = slow4d.reshape(B, C, _N_SLOW, H, W)
    fast = fast4d.reshape(B, C, _N_FAST, H, W)
    return (slow, fast)


# speedup vs baseline: 1.0398x; 1.0325x over previous
import functools

import jax
import jax.numpy as jnp
from jax import lax
from jax.experimental import pallas as pl
from jax.experimental.pallas import tpu as pltpu
from jax.experimental.pallas import tpu_sc as plsc

_N_SLOW = 8
_N_FAST = 32
_BC = 12
_NC = 2
_NS = 16
_NW = _NC * _NS
_SLOW_PER_W = (_BC * _N_SLOW) // _NW


def _sc_slow_body(frames_hbm, slow_hbm,
                  buf0, buf1, sem_in0, sem_in1, sem_out0, sem_out1):
    cid = lax.axis_index("c")
    sid = lax.axis_index("s")
    wid = sid * _NC + cid

    def plane(j):
        s = wid * _SLOW_PER_W + j
        bc = s // _N_SLOW
        t = s % _N_SLOW
        return frames_hbm.at[bc, t * 9], slow_hbm.at[bc, t]

    bufs = (buf0, buf1)
    sems_in = (sem_in0, sem_in1)
    sems_out = (sem_out0, sem_out1)
    pending_wr = [None, None]
    pending_rd = [None, None]

    src0, _ = plane(0)
    pending_rd[0] = pltpu.make_async_copy(src0, buf0, sem_in0)
    pending_rd[0].start()
    for j in range(_SLOW_PER_W):
        k = j % 2
        kn = (j + 1) % 2
        if j + 1 < _SLOW_PER_W:
            if pending_wr[kn] is not None:
                pending_wr[kn].wait()
            src_n, _ = plane(j + 1)
            pending_rd[kn] = pltpu.make_async_copy(src_n, bufs[kn],
                                                   sems_in[kn])
            pending_rd[kn].start()
        pending_rd[k].wait()
        _, dst = plane(j)
        wr = pltpu.make_async_copy(bufs[k], dst, sems_out[k])
        wr.start()
        pending_wr[k] = wr
    pending_wr[0].wait()
    pending_wr[1].wait()


def _tc_fast_body(even_ref, odd_ref, fast_ref):
    fast_ref[:, 0] = even_ref[:, 0]
    fast_ref[:, 1] = odd_ref[:, 0]


def kernel(frames):
    B, C, T, H, W = frames.shape
    frames3d = frames.reshape(B * C, T, H, W)

    mesh = plsc.VectorSubcoreMesh(core_axis_name="c", subcore_axis_name="s",
                                  num_cores=_NC, num_subcores=_NS)
    run_slow = functools.partial(
        pl.kernel,
        out_type=jax.ShapeDtypeStruct((_BC, _N_SLOW, H, W), jnp.float32),
        mesh=mesh,
        scratch_types=[
            pltpu.VMEM((H, W), jnp.float32),
            pltpu.VMEM((H, W), jnp.float32),
            pltpu.SemaphoreType.DMA,
            pltpu.SemaphoreType.DMA,
            pltpu.SemaphoreType.DMA,
            pltpu.SemaphoreType.DMA,
        ],
        compiler_params=pltpu.CompilerParams(use_tc_tiling_on_sc=True, skip_device_barrier=True),
    )(_sc_slow_body)

    fast4d = pl.pallas_call(
        _tc_fast_body,
        grid=(_N_FAST // 2,),
        in_specs=[
            pl.BlockSpec((_BC, 1, H, W), lambda k: (0, 4 * k, 0, 0)),
            pl.BlockSpec(
                (_BC, 1, H, W),
                lambda k: (0, jnp.where(k == _N_FAST // 2 - 1,
                                        2 * _N_FAST - 1, 4 * k + 2), 0, 0)),
        ],
        out_specs=pl.BlockSpec((_BC, 2, H, W), lambda k: (0, k, 0, 0)),
        out_shape=jax.ShapeDtypeStruct((_BC, _N_FAST, H, W), jnp.float32),
    )(frames3d, frames3d)
    slow4d = run_slow(frames3d)

    slow = slow4d.reshape(B, C, _N_SLOW, H, W)
    fast = fast4d.reshape(B, C, _N_FAST, H, W)
    return (slow, fast)


# speedup vs baseline: 1.0511x; 1.0109x over previous
import functools

import jax
import jax.numpy as jnp
from jax import lax
from jax.experimental import pallas as pl
from jax.experimental.pallas import tpu as pltpu
from jax.experimental.pallas import tpu_sc as plsc

_N_SLOW = 8
_N_FAST = 32
_BC = 12
_NC = 2
_NS = 16
_NW = _NC * _NS
_SLOW_PER_W = (_BC * _N_SLOW) // _NW


def _sc_slow_body(frames_hbm, slow_hbm,
                  buf0, buf1, sem_in0, sem_in1, sem_out0, sem_out1):
    cid = lax.axis_index("c")
    sid = lax.axis_index("s")
    wid = sid * _NC + cid

    def plane(j):
        s = wid * _SLOW_PER_W + j
        bc = s // _N_SLOW
        t = s % _N_SLOW
        return frames_hbm.at[bc, t * 9], slow_hbm.at[bc, t]

    bufs = (buf0, buf1)
    sems_in = (sem_in0, sem_in1)
    sems_out = (sem_out0, sem_out1)
    pending_wr = [None, None]
    pending_rd = [None, None]

    src0, _ = plane(0)
    pending_rd[0] = pltpu.make_async_copy(src0, buf0, sem_in0)
    pending_rd[0].start()
    for j in range(_SLOW_PER_W):
        k = j % 2
        kn = (j + 1) % 2
        if j + 1 < _SLOW_PER_W:
            if pending_wr[kn] is not None:
                pending_wr[kn].wait()
            src_n, _ = plane(j + 1)
            pending_rd[kn] = pltpu.make_async_copy(src_n, bufs[kn],
                                                   sems_in[kn])
            pending_rd[kn].start()
        pending_rd[k].wait()
        _, dst = plane(j)
        wr = pltpu.make_async_copy(bufs[k], dst, sems_out[k])
        wr.start()
        pending_wr[k] = wr
    pending_wr[0].wait()
    pending_wr[1].wait()


def _tc_fast_body(in0, in1, in2, in3, fast_ref):
    for s, ref in enumerate((in0, in1, in2, in3)):
        fast_ref[:, s] = ref[:, 0]


def kernel(frames):
    B, C, T, H, W = frames.shape
    frames3d = frames.reshape(B * C, T, H, W)

    mesh = plsc.VectorSubcoreMesh(core_axis_name="c", subcore_axis_name="s",
                                  num_cores=_NC, num_subcores=_NS)
    run_slow = functools.partial(
        pl.kernel,
        out_type=jax.ShapeDtypeStruct((_BC, _N_SLOW, H, W), jnp.float32),
        mesh=mesh,
        scratch_types=[
            pltpu.VMEM((H, W), jnp.float32),
            pltpu.VMEM((H, W), jnp.float32),
            pltpu.SemaphoreType.DMA,
            pltpu.SemaphoreType.DMA,
            pltpu.SemaphoreType.DMA,
            pltpu.SemaphoreType.DMA,
        ],
        compiler_params=pltpu.CompilerParams(use_tc_tiling_on_sc=True, skip_device_barrier=True),
    )(_sc_slow_body)

    fast4d = pl.pallas_call(
        _tc_fast_body,
        grid=(_N_FAST // 4,),
        in_specs=[
            pl.BlockSpec((_BC, 1, H, W), lambda k: (0, 8 * k, 0, 0)),
            pl.BlockSpec((_BC, 1, H, W), lambda k: (0, 8 * k + 2, 0, 0)),
            pl.BlockSpec((_BC, 1, H, W), lambda k: (0, 8 * k + 4, 0, 0)),
            pl.BlockSpec(
                (_BC, 1, H, W),
                lambda k: (0, jnp.where(k == _N_FAST // 4 - 1,
                                        2 * _N_FAST - 1, 8 * k + 6), 0, 0)),
        ],
        out_specs=pl.BlockSpec((_BC, 4, H, W), lambda k: (0, k, 0, 0)),
        out_shape=jax.ShapeDtypeStruct((_BC, _N_FAST, H, W), jnp.float32),
    )(frames3d, frames3d, frames3d, frames3d)
    slow4d = run_slow(frames3d)

    slow = slow4d.reshape(B, C, _N_SLOW, H, W)
    fast = fast4d.reshape(B, C, _N_FAST, H, W)
    return (slow, fast)
